# Initial kernel scaffold; baseline (speedup 1.0000x reference)
import jax
import jax.numpy as jnp
from jax.experimental import pallas as pl


def kernel(sentence, emb, w_ih0, w_hh0, b_ih0, b_hh0, w_ih1, w_hh1, b_ih1, b_hh1, W_dec, b_dec):
    raise NotImplementedError("write your pallas kernel here")


# speedup vs baseline: 3.6909x; 3.6909x over previous
import functools

import jax
import jax.numpy as jnp
from jax import lax
from jax.experimental import pallas as pl

B = 1024
T = 80
H = 256
DICT = 80
G = 4 * H


def _lstm_body(sent_ref, emb_ref, w_ih0t_ref, w_hh0t_ref, bias0_ref,
               w_ih1t_ref, w_hh1t_ref, bias1_ref, w_dect_ref, b_dec_ref,
               out_ref):
    f32 = jnp.float32
    table0 = jnp.dot(emb_ref[...], w_ih0t_ref[...],
                     preferred_element_type=f32) + bias0_ref[...]

    vocab_iota = lax.broadcasted_iota(jnp.int32, (DICT, B), 0)

    def gates(g):
        i = g[:, 0 * H:1 * H]
        f = g[:, 1 * H:2 * H]
        gg = g[:, 2 * H:3 * H]
        o = g[:, 3 * H:4 * H]
        return i, f, gg, o

    def step(t, carry):
        h0, c0, h1, c1 = carry
        row = sent_ref[pl.ds(t, 1), :]
        onehot_t = (row == vocab_iota).astype(f32)
        g0 = lax.dot_general(onehot_t, table0,
                             (((0,), (0,)), ((), ())),
                             preferred_element_type=f32)
        g0 = g0 + jnp.dot(h0, w_hh0t_ref[...], preferred_element_type=f32)
        i0, f0, gg0, o0 = gates(g0)
        c0 = jax.nn.sigmoid(f0) * c0 + jax.nn.sigmoid(i0) * jnp.tanh(gg0)
        h0 = jax.nn.sigmoid(o0) * jnp.tanh(c0)

        g1 = (jnp.dot(h0, w_ih1t_ref[...], preferred_element_type=f32)
              + jnp.dot(h1, w_hh1t_ref[...], preferred_element_type=f32)
              + bias1_ref[...])
        i1, f1, gg1, o1 = gates(g1)
        c1 = jax.nn.sigmoid(f1) * c1 + jax.nn.sigmoid(i1) * jnp.tanh(gg1)
        h1 = jax.nn.sigmoid(o1) * jnp.tanh(c1)
        return h0, c0, h1, c1

    zeros = jnp.zeros((B, H), f32)
    h0, c0, h1, c1 = lax.fori_loop(0, T, step, (zeros, zeros, zeros, zeros))
    out_ref[...] = (jnp.dot(h1, w_dect_ref[...], preferred_element_type=f32)
                    + b_dec_ref[...])


@functools.partial(jax.jit, static_argnums=())
def kernel(sentence, emb, w_ih0, w_hh0, b_ih0, b_hh0,
           w_ih1, w_hh1, b_ih1, b_hh1, W_dec, b_dec):
    sent_t = jnp.transpose(sentence.astype(jnp.int32), (1, 0))
    bias0 = (b_ih0 + b_hh0).reshape(1, G)
    bias1 = (b_ih1 + b_hh1).reshape(1, G)
    return pl.pallas_call(
        _lstm_body,
        out_shape=jax.ShapeDtypeStruct((B, DICT), jnp.float32),
    )(sent_t, emb, w_ih0.T, w_hh0.T, bias0,
      w_ih1.T, w_hh1.T, bias1, W_dec.T, b_dec.reshape(1, DICT))
